# baseline (speedup 1.0000x reference)
import functools

import jax
import jax.numpy as jnp
from jax import lax
from jax.experimental import pallas as pl
from jax.experimental.pallas import tpu as pltpu

_EPS = 1e-5


def _bn_scale_shift(pst_ref, g_ref, b_ref, count, width):
    c = g_ref.shape[1]
    s = jnp.sum(pst_ref[...], axis=0)
    acc = s[:, 0:c]
    for k in range(1, width):
        acc = acc + s[:, k * c:(k + 1) * c]
    mean = acc[0:1] / count
    var = jnp.maximum(acc[1:2] / count - mean * mean, 0.0)
    scale = g_ref[...] * lax.rsqrt(var + _EPS)
    shift = b_ref[...] - mean * scale
    st = jnp.concatenate([scale, shift], axis=0)
    st = jnp.concatenate([st] * width, axis=1)
    return st[0:1], st[1:2]


def _build_bands(w_ref, a_scr, ww):
    cc = w_ref.shape[2]
    a_scr[...] = jnp.zeros_like(a_scr)
    for dy in range(3):
        for dx in range(3):
            blk = w_ref[dy, dx].astype(jnp.bfloat16)
            for xx in range(ww):
                xin = xx + dx - 1
                if 0 <= xin < ww:
                    col = (xx // 4) * 12 * cc + dy * 4 * cc + (xx % 4) * cc
                    a_scr[(2 + xin) * cc:(3 + xin) * cc, col:col + cc] = blk


def _banded_conv(yb, y_scr, a_scr, st_ref, z_ref, hh, ww):
    bh, wc = yb.shape
    cc = wc // ww
    bb = bh // hh
    y_scr[:, 2 * cc:2 * cc + wc] = yb
    rid = lax.broadcasted_iota(jnp.int32, (bh, 1), 0) % hh
    m0 = rid != hh - 1
    m2 = rid != 0
    gw = 4 * cc
    for g in range(ww // 4):
        k0 = g * gw
        p = jnp.dot(y_scr[:, k0:k0 + 2 * gw],
                    a_scr[k0:k0 + 2 * gw, 3 * k0:3 * k0 + 3 * gw],
                    preferred_element_type=jnp.float32)
        u0 = jnp.where(m0, p[:, :gw], 0.0)
        u2 = jnp.where(m2, p[:, 2 * gw:], 0.0)
        zg = (p[:, gw:2 * gw]
              + pltpu.roll(u0, 1, axis=0)
              + pltpu.roll(u2, bh - 1, axis=0))
        st_ref[0, 0:1, k0:k0 + gw] += jnp.sum(zg, axis=0, keepdims=True)
        st_ref[0, 1:2, k0:k0 + gw] += jnp.sum(zg * zg, axis=0, keepdims=True)
        z_ref[:, :, k0:k0 + gw] = zg.astype(z_ref.dtype).reshape(bb, hh, gw)


def _conv1_kernel(x_ref, w_ref, z_ref, st_ref, a_scr, y_scr, *, hh, ww):
    @pl.when(pl.program_id(1) == 0)
    def _init():
        st_ref[...] = jnp.zeros_like(st_ref)
        _build_bands(w_ref, a_scr, ww)
        y_scr[...] = jnp.zeros_like(y_scr)

    bb, _, wc = x_ref.shape
    yb = x_ref[...].reshape(bb * hh, wc)
    _banded_conv(yb, y_scr, a_scr, st_ref, z_ref, hh, ww)


def _conv2_kernel(x_ref, w_ref, pst_ref, g_ref, b_ref, z_ref, st_ref, a_scr,
                  y_scr, *, hh, ww, count):
    @pl.when(pl.program_id(1) == 0)
    def _init():
        st_ref[...] = jnp.zeros_like(st_ref)
        _build_bands(w_ref, a_scr, ww)
        y_scr[...] = jnp.zeros_like(y_scr)

    bb, _, wc = x_ref.shape
    scale, shift = _bn_scale_shift(pst_ref, g_ref, b_ref, count, ww)
    y = x_ref[...].reshape(bb * hh, wc).astype(jnp.float32) * scale + shift
    yb = jnp.maximum(y, 0.0).astype(jnp.bfloat16)
    _banded_conv(yb, y_scr, a_scr, st_ref, z_ref, hh, ww)


def _residual_kernel(z_ref, x_ref, pst_ref, g_ref, b_ref, o_ref, *,
                     ww, count):
    scale, shift = _bn_scale_shift(pst_ref, g_ref, b_ref, count, ww)
    o_ref[...] = jnp.maximum(
        z_ref[...].astype(jnp.float32) * scale + shift
        + x_ref[...].astype(jnp.float32), 0.0)


def _pick_block(n, candidates):
    for b in candidates:
        if n % b == 0:
            return b
    return 1


def _stats_spec(wc):
    return pl.BlockSpec((1, 2, wc), lambda i, j: (i, 0, 0))


def _conv1_pass(x_dense, w_hwio):
    n, hh, wc = x_dense.shape
    ww = wc // w_hwio.shape[2]
    cores = 2 if n % 2 == 0 else 1
    bb = _pick_block(n // cores, (16, 8, 4, 2, 1))
    inner = n // (cores * bb)
    img = pl.BlockSpec((bb, hh, wc), lambda i, j, g=inner: (i * g + j, 0, 0))
    return pl.pallas_call(
        functools.partial(_conv1_kernel, hh=hh, ww=ww),
        grid=(cores, inner),
        in_specs=[img, pl.BlockSpec(w_hwio.shape, lambda i, j: (0,) * 4)],
        out_specs=(img, _stats_spec(wc)),
        out_shape=(
            jax.ShapeDtypeStruct((n, hh, wc), jnp.bfloat16),
            jax.ShapeDtypeStruct((cores, 2, wc), jnp.float32),
        ),
        scratch_shapes=[pltpu.VMEM((wc + 4 * (wc // ww), 3 * wc), jnp.bfloat16),
                        pltpu.VMEM((bb * hh, wc + 4 * (wc // ww)), jnp.bfloat16)],
        compiler_params=pltpu.CompilerParams(
            dimension_semantics=("parallel", "arbitrary"),
            vmem_limit_bytes=64 * 1024 * 1024,
        ),
    )(x_dense, w_hwio)


def _conv2_pass(z1, w_hwio, st1, g, b, count):
    n, hh, wc = z1.shape
    ww = wc // w_hwio.shape[2]
    cores = st1.shape[0]
    bb = _pick_block(n // cores, (16, 8, 4, 2, 1))
    inner = n // (cores * bb)
    img = pl.BlockSpec((bb, hh, wc), lambda i, j, g=inner: (i * g + j, 0, 0))
    vec = pl.BlockSpec((1, g.shape[1]), lambda i, j: (0, 0))
    return pl.pallas_call(
        functools.partial(_conv2_kernel, hh=hh, ww=ww, count=count),
        grid=(cores, inner),
        in_specs=[
            img,
            pl.BlockSpec(w_hwio.shape, lambda i, j: (0,) * 4),
            pl.BlockSpec(st1.shape, lambda i, j: (0, 0, 0)),
            vec, vec,
        ],
        out_specs=(img, _stats_spec(wc)),
        out_shape=(
            jax.ShapeDtypeStruct((n, hh, wc), jnp.bfloat16),
            jax.ShapeDtypeStruct((cores, 2, wc), jnp.float32),
        ),
        scratch_shapes=[pltpu.VMEM((wc + 4 * (wc // ww), 3 * wc), jnp.bfloat16),
                        pltpu.VMEM((bb * hh, wc + 4 * (wc // ww)), jnp.bfloat16)],
        compiler_params=pltpu.CompilerParams(
            dimension_semantics=("parallel", "arbitrary"),
            vmem_limit_bytes=64 * 1024 * 1024,
        ),
    )(z1, w_hwio, st1, g, b)


def _residual_pass(z2, x_dense, st2, g, b, count, ww):
    n, hh, wc = z2.shape
    bb = _pick_block(n, (32, 16, 8, 4, 2, 1))
    img = pl.BlockSpec((bb, hh, wc), lambda i: (i, 0, 0))
    vec = pl.BlockSpec((1, g.shape[1]), lambda i: (0, 0))
    return pl.pallas_call(
        functools.partial(_residual_kernel, ww=ww, count=count),
        grid=(n // bb,),
        in_specs=[
            img, img,
            pl.BlockSpec(st2.shape, lambda i: (0, 0, 0)),
            vec, vec,
        ],
        out_specs=img,
        out_shape=jax.ShapeDtypeStruct((n, hh, wc), jnp.float32),
        compiler_params=pltpu.CompilerParams(
            dimension_semantics=("parallel",),
            vmem_limit_bytes=64 * 1024 * 1024,
        ),
    )(z2, x_dense, st2, g, b)


@jax.jit
def kernel(x, w1, g1, b1, w2, g2, b2):
    n, c, h, w = x.shape
    wc = w * c
    count = float(n * h * w)

    x_bf = jnp.transpose(x.astype(jnp.bfloat16), (0, 2, 3, 1)).reshape(n, h, wc)
    g1r = g1.astype(jnp.float32).reshape(1, c)
    b1r = b1.astype(jnp.float32).reshape(1, c)
    g2r = g2.astype(jnp.float32).reshape(1, c)
    b2r = b2.astype(jnp.float32).reshape(1, c)

    z1, st1 = _conv1_pass(x_bf, w1.astype(jnp.float32))
    z2, st2 = _conv2_pass(z1, w2.astype(jnp.float32), st1, g1r, b1r, count)
    out_dense = _residual_pass(z2, x_bf, st2, g2r, b2r, count, w)

    out = out_dense.reshape(n, h, w, c)
    return jnp.transpose(out, (0, 3, 1, 2)).astype(x.dtype)


# speedup vs baseline: 1.0469x; 1.0469x over previous
import functools

import jax
import jax.numpy as jnp
from jax import lax
from jax.experimental import pallas as pl
from jax.experimental.pallas import tpu as pltpu

_EPS = 1e-5


def _bn_scale_shift(pst_ref, g_ref, b_ref, count, width):
    c = g_ref.shape[1]
    s = jnp.sum(pst_ref[...], axis=(0, 1))
    acc = s[:, 0:c]
    for k in range(1, width):
        acc = acc + s[:, k * c:(k + 1) * c]
    mean = acc[0:1] / count
    var = jnp.maximum(acc[1:2] / count - mean * mean, 0.0)
    scale = g_ref[...] * lax.rsqrt(var + _EPS)
    shift = b_ref[...] - mean * scale
    st = jnp.concatenate([scale, shift], axis=0)
    st = jnp.concatenate([st] * width, axis=1)
    return st[0:1], st[1:2]


def _build_bands(w_ref, a_scr, ww):
    cc = w_ref.shape[2]
    a_scr[...] = jnp.zeros_like(a_scr)
    for dy in range(3):
        for dx in range(3):
            blk = w_ref[dy, dx].astype(jnp.bfloat16)
            for xx in range(ww):
                xin = xx + dx - 1
                if 0 <= xin < ww:
                    col = (xx // 4) * 12 * cc + dy * 4 * cc + (xx % 4) * cc
                    a_scr[(2 + xin) * cc:(3 + xin) * cc, col:col + cc] = blk


def _banded_conv(yb, y_scr, a_scr, st_ref, z_ref, hh, ww):
    bh, wc = yb.shape
    cc = wc // ww
    bb = bh // hh
    y_scr[:, 2 * cc:2 * cc + wc] = yb
    rid = lax.broadcasted_iota(jnp.int32, (bh, 1), 0) % hh
    m0 = rid != hh - 1
    m2 = rid != 0
    gw = 4 * cc
    for g in range(ww // 4):
        k0 = g * gw
        p = jnp.dot(y_scr[:, k0:k0 + 2 * gw],
                    a_scr[k0:k0 + 2 * gw, 3 * k0:3 * k0 + 3 * gw],
                    preferred_element_type=jnp.float32)
        u0 = jnp.where(m0, p[:, :gw], 0.0)
        u2 = jnp.where(m2, p[:, 2 * gw:], 0.0)
        zg = (p[:, gw:2 * gw]
              + pltpu.roll(u0, 1, axis=0)
              + pltpu.roll(u2, bh - 1, axis=0))
        st_ref[0, 0, 0:1, k0:k0 + gw] = jnp.sum(zg, axis=0, keepdims=True)
        st_ref[0, 0, 1:2, k0:k0 + gw] = jnp.sum(zg * zg, axis=0, keepdims=True)
        z_ref[:, :, k0:k0 + gw] = zg.astype(z_ref.dtype).reshape(bb, hh, gw)


def _conv1_kernel(x_ref, w_ref, z_ref, st_ref, a_scr, y_scr, *, hh, ww):
    @pl.when(pl.program_id(1) == 0)
    def _init():
        _build_bands(w_ref, a_scr, ww)
        y_scr[...] = jnp.zeros_like(y_scr)

    bb, _, wc = x_ref.shape
    yb = x_ref[...].reshape(bb * hh, wc)
    _banded_conv(yb, y_scr, a_scr, st_ref, z_ref, hh, ww)


def _conv2_kernel(x_ref, w_ref, pst_ref, g_ref, b_ref, z_ref, st_ref, a_scr,
                  y_scr, *, hh, ww, count):
    @pl.when(pl.program_id(1) == 0)
    def _init():
        _build_bands(w_ref, a_scr, ww)
        y_scr[...] = jnp.zeros_like(y_scr)

    bb, _, wc = x_ref.shape
    scale, shift = _bn_scale_shift(pst_ref, g_ref, b_ref, count, ww)
    y = x_ref[...].reshape(bb * hh, wc).astype(jnp.float32) * scale + shift
    yb = jnp.maximum(y, 0.0).astype(jnp.bfloat16)
    _banded_conv(yb, y_scr, a_scr, st_ref, z_ref, hh, ww)


def _residual_kernel(z_ref, x_ref, pst_ref, g_ref, b_ref, o_ref, *,
                     ww, count):
    scale, shift = _bn_scale_shift(pst_ref, g_ref, b_ref, count, ww)
    o_ref[...] = jnp.maximum(
        z_ref[...].astype(jnp.float32) * scale + shift
        + x_ref[...].astype(jnp.float32), 0.0)


def _pick_block(n, candidates):
    for b in candidates:
        if n % b == 0:
            return b
    return 1


def _stats_spec(wc):
    return pl.BlockSpec((1, 1, 2, wc), lambda i, j: (i, j, 0, 0))


def _conv1_pass(x_dense, w_hwio):
    n, hh, wc = x_dense.shape
    ww = wc // w_hwio.shape[2]
    cores = 2 if n % 2 == 0 else 1
    bb = _pick_block(n // cores, (32, 16, 8, 4, 2, 1))
    inner = n // (cores * bb)
    img = pl.BlockSpec((bb, hh, wc), lambda i, j, g=inner: (i * g + j, 0, 0))
    return pl.pallas_call(
        functools.partial(_conv1_kernel, hh=hh, ww=ww),
        grid=(cores, inner),
        in_specs=[img, pl.BlockSpec(w_hwio.shape, lambda i, j: (0,) * 4)],
        out_specs=(img, _stats_spec(wc)),
        out_shape=(
            jax.ShapeDtypeStruct((n, hh, wc), jnp.bfloat16),
            jax.ShapeDtypeStruct((cores, inner, 2, wc), jnp.float32),
        ),
        scratch_shapes=[pltpu.VMEM((wc + 4 * (wc // ww), 3 * wc), jnp.bfloat16),
                        pltpu.VMEM((bb * hh, wc + 4 * (wc // ww)), jnp.bfloat16)],
        compiler_params=pltpu.CompilerParams(
            dimension_semantics=("parallel", "arbitrary"),
            vmem_limit_bytes=64 * 1024 * 1024,
        ),
    )(x_dense, w_hwio)


def _conv2_pass(z1, w_hwio, st1, g, b, count):
    n, hh, wc = z1.shape
    ww = wc // w_hwio.shape[2]
    cores = st1.shape[0]
    bb = _pick_block(n // cores, (32, 16, 8, 4, 2, 1))
    inner = n // (cores * bb)
    img = pl.BlockSpec((bb, hh, wc), lambda i, j, g=inner: (i * g + j, 0, 0))
    vec = pl.BlockSpec((1, g.shape[1]), lambda i, j: (0, 0))
    return pl.pallas_call(
        functools.partial(_conv2_kernel, hh=hh, ww=ww, count=count),
        grid=(cores, inner),
        in_specs=[
            img,
            pl.BlockSpec(w_hwio.shape, lambda i, j: (0,) * 4),
            pl.BlockSpec(st1.shape, lambda i, j: (0, 0, 0, 0)),
            vec, vec,
        ],
        out_specs=(img, _stats_spec(wc)),
        out_shape=(
            jax.ShapeDtypeStruct((n, hh, wc), jnp.bfloat16),
            jax.ShapeDtypeStruct((cores, inner, 2, wc), jnp.float32),
        ),
        scratch_shapes=[pltpu.VMEM((wc + 4 * (wc // ww), 3 * wc), jnp.bfloat16),
                        pltpu.VMEM((bb * hh, wc + 4 * (wc // ww)), jnp.bfloat16)],
        compiler_params=pltpu.CompilerParams(
            dimension_semantics=("parallel", "arbitrary"),
            vmem_limit_bytes=64 * 1024 * 1024,
        ),
    )(z1, w_hwio, st1, g, b)


def _residual_pass(z2, x_dense, st2, g, b, count, ww):
    n, hh, wc = z2.shape
    bb = _pick_block(n, (32, 16, 8, 4, 2, 1))
    img = pl.BlockSpec((bb, hh, wc), lambda i: (i, 0, 0))
    vec = pl.BlockSpec((1, g.shape[1]), lambda i: (0, 0))
    return pl.pallas_call(
        functools.partial(_residual_kernel, ww=ww, count=count),
        grid=(n // bb,),
        in_specs=[
            img, img,
            pl.BlockSpec(st2.shape, lambda i: (0, 0, 0, 0)),
            vec, vec,
        ],
        out_specs=img,
        out_shape=jax.ShapeDtypeStruct((n, hh, wc), jnp.float32),
        compiler_params=pltpu.CompilerParams(
            dimension_semantics=("parallel",),
            vmem_limit_bytes=64 * 1024 * 1024,
        ),
    )(z2, x_dense, st2, g, b)


@jax.jit
def kernel(x, w1, g1, b1, w2, g2, b2):
    n, c, h, w = x.shape
    wc = w * c
    count = float(n * h * w)

    x_bf = jnp.transpose(x.astype(jnp.bfloat16), (0, 2, 3, 1)).reshape(n, h, wc)
    g1r = g1.astype(jnp.float32).reshape(1, c)
    b1r = b1.astype(jnp.float32).reshape(1, c)
    g2r = g2.astype(jnp.float32).reshape(1, c)
    b2r = b2.astype(jnp.float32).reshape(1, c)

    z1, st1 = _conv1_pass(x_bf, w1.astype(jnp.float32))
    z2, st2 = _conv2_pass(z1, w2.astype(jnp.float32), st1, g1r, b1r, count)
    out_dense = _residual_pass(z2, x_bf, st2, g2r, b2r, count, w)

    out = out_dense.reshape(n, h, w, c)
    return jnp.transpose(out, (0, 3, 1, 2)).astype(x.dtype)
